# baseline (speedup 1.0000x reference)
import functools

import jax
import jax.numpy as jnp
from jax import lax
from jax.experimental import pallas as pl
from jax.experimental.pallas import tpu as pltpu
from jax.experimental.pallas import tpu_sc as plsc

_NC = 2
_NS = 16
_LANES = 16


def _sc_aggregate(x2, alpha_r, idxi_r, idxj_r, n_nodes, fh,
                  n_super, cps, chunk):
    rows_per_tile = n_nodes // _NS
    n_z = 5
    zrows = rows_per_tile // n_z
    mesh = plsc.VectorSubcoreMesh(core_axis_name="c", subcore_axis_name="s")

    @functools.partial(
        pl.kernel,
        out_type=jax.ShapeDtypeStruct((_NC * _NS, rows_per_tile, fh),
                                      jnp.float32),
        mesh=mesh,
        compiler_params=pltpu.CompilerParams(use_tc_tiling_on_sc=False),
        scratch_types=[
            pltpu.VMEM((cps, chunk), jnp.int32),
            pltpu.VMEM((cps, chunk), jnp.int32),
            pltpu.VMEM((cps, chunk), jnp.float32),
            pltpu.VMEM((chunk, fh), jnp.float32),
            pltpu.VMEM((chunk, fh), jnp.float32),
            pltpu.VMEM((chunk, fh), jnp.float32),
            pltpu.VMEM((chunk, fh), jnp.float32),
            pltpu.VMEM((zrows, fh), jnp.float32),
            pltpu.VMEM_SHARED((n_nodes, fh), jnp.float32),
            pltpu.SemaphoreType.DMA,
            pltpu.SemaphoreType.DMA,
            pltpu.SemaphoreType.DMA,
            pltpu.SemaphoreType.DMA,
            pltpu.SemaphoreType.DMA,
        ],
    )
    def body(x_hbm, alpha_hbm, idxi_hbm, idxj_hbm, out_hbm,
             idxj_v, idxi_v, alpha_v, g0, g1, s0, s1, zbuf, acc,
             semg0, semg1, sems0, sems1, semz):
        c = lax.axis_index("c")
        s = lax.axis_index("s")
        w = c * _NS + s
        gbufs = (g0, g1)
        sbufs = (s0, s1)
        semg = (semg0, semg1)
        sems = (sems0, sems1)

        @pl.loop(0, zrows)
        def _zrow(i):
            for t in range(fh // _LANES):
                zbuf[i, pl.ds(t * _LANES, _LANES)] = jnp.zeros(
                    (_LANES,), jnp.float32)

        base = s * rows_per_tile
        for z in range(n_z):
            pltpu.async_copy(zbuf, acc.at[pl.ds(base + z * zrows, zrows)],
                             semz)
        for z in range(n_z):
            pltpu.make_async_copy(
                zbuf, acc.at[pl.ds(base + z * zrows, zrows)], semz).wait()
        plsc.subcore_barrier()

        def scale(k, gbuf, sbuf):
            @pl.loop(0, chunk // _LANES)
            def _sgrp(gg):
                av = alpha_v[k, pl.ds(gg * _LANES, _LANES)]
                for r16 in range(_LANES):
                    a = av[r16]
                    r = gg * _LANES + r16
                    for t in range(fh // _LANES):
                        sl = pl.ds(t * _LANES, _LANES)
                        sbuf[r, sl] = gbuf[r, sl] * a

        @pl.loop(0, n_super)
        def _super(u):
            pltpu.sync_copy(idxj_hbm.at[c, s, u], idxj_v)
            pltpu.sync_copy(idxi_hbm.at[s, u], idxi_v)
            pltpu.sync_copy(alpha_hbm.at[s, u], alpha_v)

            pltpu.async_copy(x_hbm.at[idxj_v.at[0]], g0, semg0)
            pltpu.async_copy(x_hbm.at[idxj_v.at[1]], g1, semg1)

            @pl.loop(0, cps // 2)
            def _pair(kk):
                for b in range(2):
                    k = kk * 2 + b
                    pltpu.make_async_copy(
                        x_hbm.at[idxj_v.at[k]], gbufs[b], semg[b]).wait()
                    @pl.when(k >= 2)
                    def _wsc():
                        pltpu.make_async_copy(
                            sbufs[b], acc.at[idxi_v.at[k]], sems[b]).wait()

                    scale(k, gbufs[b], sbufs[b])

                    @pl.when(k + 2 < cps)
                    def _gnext():
                        pltpu.async_copy(
                            x_hbm.at[idxj_v.at[k + 2]], gbufs[b], semg[b])

                    pltpu.async_copy(
                        sbufs[b], acc.at[idxi_v.at[k]], sems[b],
                        add=True)

            for b in range(2):
                k = cps - 2 + b
                pltpu.make_async_copy(
                    sbufs[b], acc.at[idxi_v.at[k]], sems[b]).wait()

        plsc.subcore_barrier()
        pltpu.sync_copy(acc.at[pl.ds(base, rows_per_tile)], out_hbm.at[w])

    return body(x2, alpha_r, idxi_r, idxj_r)


def _tc_finish(p_lo, p_hi, w_top, w_bot, n_nodes, feat, fh, block):
    def body(lo_ref, hi_ref, wt_ref, wb_ref, o_ref):
        o_ref[...] = (
            jnp.dot(lo_ref[...], wt_ref[...],
                    preferred_element_type=jnp.float32)
            + jnp.dot(hi_ref[...], wb_ref[...],
                      preferred_element_type=jnp.float32))

    return pl.pallas_call(
        body,
        grid=(n_nodes // block,),
        in_specs=[
            pl.BlockSpec((block, fh), lambda i: (i, 0)),
            pl.BlockSpec((block, fh), lambda i: (i, 0)),
            pl.BlockSpec((fh, feat), lambda i: (0, 0)),
            pl.BlockSpec((fh, feat), lambda i: (0, 0)),
        ],
        out_specs=pl.BlockSpec((block, feat), lambda i: (i, 0)),
        out_shape=jax.ShapeDtypeStruct((n_nodes, feat), jnp.float32),
    )(p_lo, p_hi, w_top, w_bot)


def kernel(x, alpha_ij, idx_i, idx_j, W):
    n_nodes, feat = x.shape
    n_edges = alpha_ij.shape[0]
    fh = feat // _NC
    chunk = 80
    n_super, cps = 5, 50
    assert _NS * n_super * cps * chunk == n_edges

    idx_i32 = idx_i.astype(jnp.int32)
    idx_j32 = idx_j.astype(jnp.int32)
    eshape = (_NS, n_super, cps, chunk)
    idxi_r = idx_i32.reshape(eshape)
    alpha_r = alpha_ij.astype(jnp.float32).reshape(eshape)
    idxj_r = jnp.stack(
        [idx_j32.reshape(eshape), (idx_j32 + n_nodes).reshape(eshape)])

    xf = x.astype(jnp.float32)
    x2 = jnp.concatenate([xf[:, :fh], xf[:, fh:]], axis=0)

    partial = _sc_aggregate(x2, alpha_r, idxi_r, idxj_r,
                            n_nodes, fh, n_super, cps, chunk)
    partial = partial.reshape(_NC, n_nodes, fh)
    Wf = W.astype(jnp.float32)
    return _tc_finish(partial[0], partial[1], Wf[:fh], Wf[fh:],
                      n_nodes, feat, fh, 400)


# speedup vs baseline: 1.0332x; 1.0332x over previous
import functools

import jax
import jax.numpy as jnp
from jax import lax
from jax.experimental import pallas as pl
from jax.experimental.pallas import tpu as pltpu
from jax.experimental.pallas import tpu_sc as plsc

_NC = 2
_NS = 16
_LANES = 16


def _sc_aggregate(x, alpha_r, idxi_r, idxj_r, n_nodes, feat,
                  n_super, cps, chunk):
    rows_per_tile = n_nodes // _NS
    mesh = plsc.VectorSubcoreMesh(core_axis_name="c", subcore_axis_name="s")

    @functools.partial(
        pl.kernel,
        out_type=jax.ShapeDtypeStruct((_NC * _NS, rows_per_tile, feat),
                                      jnp.float32),
        mesh=mesh,
        scratch_types=[
            pltpu.VMEM((cps, chunk), jnp.int32),
            pltpu.VMEM((cps, chunk), jnp.int32),
            pltpu.VMEM((cps, chunk), jnp.float32),
            pltpu.VMEM((chunk, feat), jnp.float32),
            pltpu.VMEM((chunk, feat), jnp.float32),
            pltpu.VMEM_SHARED((n_nodes, feat), jnp.float32),
            pltpu.SemaphoreType.DMA,
            pltpu.SemaphoreType.DMA,
            pltpu.SemaphoreType.DMA,
            pltpu.SemaphoreType.DMA,
        ],
    )
    def body(x_hbm, alpha_hbm, idxi_hbm, idxj_hbm, out_hbm,
             idxj_v, idxi_v, alpha_v, g0, g1, acc,
             semg0, semg1, sems0, sems1):
        c = lax.axis_index("c")
        s = lax.axis_index("s")
        w = c * _NS + s
        gbufs = (g0, g1)
        semg = (semg0, semg1)
        sems = (sems0, sems1)

        @pl.loop(0, chunk)
        def _zrow(i):
            for t in range(feat // _LANES):
                z = jnp.zeros((_LANES,), jnp.float32)
                g0[i, pl.ds(t * _LANES, _LANES)] = z

        base = s * rows_per_tile
        n_full = rows_per_tile // chunk
        rem = rows_per_tile - n_full * chunk
        for z in range(n_full):
            pltpu.async_copy(g0, acc.at[pl.ds(base + z * chunk, chunk)],
                             semg0)
        pltpu.async_copy(g0.at[pl.ds(0, rem)],
                         acc.at[pl.ds(base + n_full * chunk, rem)], semg1)
        for z in range(n_full):
            pltpu.make_async_copy(
                g0, acc.at[pl.ds(base + z * chunk, chunk)], semg0).wait()
        pltpu.make_async_copy(
            g0.at[pl.ds(0, rem)],
            acc.at[pl.ds(base + n_full * chunk, rem)], semg1).wait()
        plsc.subcore_barrier()

        def scale_half(k, g, lo, hi):
            @pl.loop(lo, hi)
            def _sgrp(gg):
                av = alpha_v[k, pl.ds(gg * _LANES, _LANES)]
                for r16 in range(_LANES):
                    a = av[r16]
                    r = gg * _LANES + r16
                    for t in range(feat // _LANES):
                        sl = pl.ds(t * _LANES, _LANES)
                        g[r, sl] = g[r, sl] * a

        n_grp = chunk // _LANES

        @pl.loop(0, n_super)
        def _super(u):
            pltpu.sync_copy(idxj_hbm.at[w, u], idxj_v)
            pltpu.sync_copy(idxi_hbm.at[w, u], idxi_v)
            pltpu.sync_copy(alpha_hbm.at[w, u], alpha_v)

            pltpu.async_copy(x_hbm.at[idxj_v.at[0]], g0, semg0)

            def process_chunk(k, b, issue_next):
                g = gbufs[b]
                pltpu.make_async_copy(
                    x_hbm.at[idxj_v.at[k]], g, semg[b]).wait()

                scale_half(k, g, 0, n_grp // 2)

                @pl.when(k >= 1)
                def _drain():
                    pltpu.make_async_copy(
                        gbufs[1 - b], acc.at[idxi_v.at[k]],
                        sems[1 - b]).wait()
                if issue_next:
                    pltpu.async_copy(
                        x_hbm.at[idxj_v.at[k + 1]], gbufs[1 - b],
                        semg[1 - b])

                scale_half(k, g, n_grp // 2, n_grp)

                pltpu.async_copy(g, acc.at[idxi_v.at[k]], sems[b],
                                 add=True)

            @pl.loop(0, cps // 2)
            def _pair(kk):
                for b in range(2):
                    process_chunk(kk * 2 + b, b, issue_next=True)

            if cps % 2:
                process_chunk(cps - 1, 0, issue_next=False)

            pltpu.make_async_copy(
                gbufs[(cps - 1) % 2], acc.at[idxi_v.at[cps - 1]],
                sems[(cps - 1) % 2]).wait()

        plsc.subcore_barrier()
        pltpu.sync_copy(acc.at[pl.ds(base, rows_per_tile)], out_hbm.at[w])

    return body(x, alpha_r, idxi_r, idxj_r)


def _tc_finish(p0, p1, W, n_nodes, feat, block):
    def body(p0_ref, p1_ref, w_ref, o_ref):
        o_ref[...] = jnp.dot(p0_ref[...] + p1_ref[...], w_ref[...],
                             preferred_element_type=jnp.float32)

    return pl.pallas_call(
        body,
        grid=(n_nodes // block,),
        in_specs=[
            pl.BlockSpec((block, feat), lambda i: (i, 0)),
            pl.BlockSpec((block, feat), lambda i: (i, 0)),
            pl.BlockSpec((feat, feat), lambda i: (0, 0)),
        ],
        out_specs=pl.BlockSpec((block, feat), lambda i: (i, 0)),
        out_shape=jax.ShapeDtypeStruct((n_nodes, feat), jnp.float32),
    )(p0, p1, W)


def kernel(x, alpha_ij, idx_i, idx_j, W):
    n_nodes, feat = x.shape
    n_edges = alpha_ij.shape[0]
    nw = _NC * _NS
    chunk = 80
    n_super, cps = 5, 25
    assert nw * n_super * cps * chunk == n_edges

    shape = (nw, n_super, cps, chunk)
    idxi_r = idx_i.astype(jnp.int32).reshape(shape)
    idxj_r = idx_j.astype(jnp.int32).reshape(shape)
    alpha_r = alpha_ij.astype(jnp.float32).reshape(shape)

    partial = _sc_aggregate(x.astype(jnp.float32), alpha_r, idxi_r, idxj_r,
                            n_nodes, feat, n_super, cps, chunk)
    partial = partial.reshape(_NC, n_nodes, feat)
    return _tc_finish(partial[0], partial[1],
                      W.astype(jnp.float32), n_nodes, feat, 400)


# speedup vs baseline: 1.1142x; 1.0784x over previous
import functools

import jax
import jax.numpy as jnp
from jax import lax
from jax.experimental import pallas as pl
from jax.experimental.pallas import tpu as pltpu
from jax.experimental.pallas import tpu_sc as plsc

_NC = 2
_NS = 16
_LANES = 16


def _sc_aggregate(x, alpha_r, idxi_r, idxj_r, n_nodes, feat,
                  n_super, cps, chunk):
    rows_per_tile = n_nodes // _NS
    mesh = plsc.VectorSubcoreMesh(core_axis_name="c", subcore_axis_name="s")

    @functools.partial(
        pl.kernel,
        out_type=jax.ShapeDtypeStruct((_NC * _NS, rows_per_tile, feat),
                                      jnp.float32),
        mesh=mesh,
        scratch_types=[
            pltpu.VMEM((cps, chunk), jnp.int32),
            pltpu.VMEM((cps, chunk), jnp.int32),
            pltpu.VMEM((cps, chunk), jnp.float32),
            pltpu.VMEM((chunk, feat), jnp.float32),
            pltpu.VMEM((chunk, feat), jnp.float32),
            pltpu.VMEM_SHARED((n_nodes, feat), jnp.float32),
            pltpu.SemaphoreType.DMA,
            pltpu.SemaphoreType.DMA,
            pltpu.SemaphoreType.DMA,
            pltpu.SemaphoreType.DMA,
        ],
    )
    def body(x_hbm, alpha_hbm, idxi_hbm, idxj_hbm, out_hbm,
             idxj_v, idxi_v, alpha_v, g0, g1, acc,
             semg0, semg1, sems0, sems1):
        c = lax.axis_index("c")
        s = lax.axis_index("s")
        w = c * _NS + s
        gbufs = (g0, g1)
        semg = (semg0, semg1)
        sems = (sems0, sems1)

        @pl.loop(0, chunk)
        def _zrow(i):
            for t in range(feat // _LANES):
                z = jnp.zeros((_LANES,), jnp.float32)
                g0[i, pl.ds(t * _LANES, _LANES)] = z

        base = s * rows_per_tile
        n_full = rows_per_tile // chunk
        rem = rows_per_tile - n_full * chunk
        for z in range(n_full):
            pltpu.async_copy(g0, acc.at[pl.ds(base + z * chunk, chunk)],
                             semg0)
        pltpu.async_copy(g0.at[pl.ds(0, rem)],
                         acc.at[pl.ds(base + n_full * chunk, rem)], semg1)
        for z in range(n_full):
            pltpu.make_async_copy(
                g0, acc.at[pl.ds(base + z * chunk, chunk)], semg0).wait()
        pltpu.make_async_copy(
            g0.at[pl.ds(0, rem)],
            acc.at[pl.ds(base + n_full * chunk, rem)], semg1).wait()
        plsc.subcore_barrier()

        n_grp = chunk // _LANES

        def scale(k, g):
            @plsc.parallel_loop(0, n_grp)
            def _sgrp(gg):
                av = alpha_v[k, pl.ds(gg * _LANES, _LANES)]
                for r16 in range(_LANES):
                    a = av[r16]
                    r = gg * _LANES + r16
                    for t in range(feat // _LANES):
                        sl = pl.ds(t * _LANES, _LANES)
                        g[r, sl] = g[r, sl] * a

        @pl.loop(0, n_super)
        def _super(u):
            pltpu.sync_copy(idxj_hbm.at[w, u], idxj_v)
            pltpu.sync_copy(idxi_hbm.at[w, u], idxi_v)
            pltpu.sync_copy(alpha_hbm.at[w, u], alpha_v)

            pltpu.async_copy(x_hbm.at[idxj_v.at[0]], g0, semg0)

            def process_chunk(k, b, issue_next):
                g = gbufs[b]
                pltpu.make_async_copy(
                    x_hbm.at[idxj_v.at[k]], g, semg[b]).wait()

                if issue_next:
                    pltpu.async_copy(
                        x_hbm.at[idxj_v.at[k + 1]], gbufs[1 - b],
                        semg[1 - b])

                scale(k, g)

                pltpu.sync_copy(g, acc.at[idxi_v.at[k]], add=True)

            @pl.loop(0, cps // 2)
            def _pair(kk):
                for b in range(2):
                    process_chunk(kk * 2 + b, b, issue_next=True)

            if cps % 2:
                process_chunk(cps - 1, 0, issue_next=False)

        plsc.subcore_barrier()
        pltpu.sync_copy(acc.at[pl.ds(base, rows_per_tile)], out_hbm.at[w])

    return body(x, alpha_r, idxi_r, idxj_r)


def _tc_finish(p0, p1, W, n_nodes, feat, block):
    def body(p0_ref, p1_ref, w_ref, o_ref):
        o_ref[...] = jnp.dot(p0_ref[...] + p1_ref[...], w_ref[...],
                             preferred_element_type=jnp.float32)

    return pl.pallas_call(
        body,
        grid=(n_nodes // block,),
        in_specs=[
            pl.BlockSpec((block, feat), lambda i: (i, 0)),
            pl.BlockSpec((block, feat), lambda i: (i, 0)),
            pl.BlockSpec((feat, feat), lambda i: (0, 0)),
        ],
        out_specs=pl.BlockSpec((block, feat), lambda i: (i, 0)),
        out_shape=jax.ShapeDtypeStruct((n_nodes, feat), jnp.float32),
    )(p0, p1, W)


def kernel(x, alpha_ij, idx_i, idx_j, W):
    n_nodes, feat = x.shape
    n_edges = alpha_ij.shape[0]
    nw = _NC * _NS
    chunk = 80
    n_super, cps = 5, 25
    assert nw * n_super * cps * chunk == n_edges

    shape = (nw, n_super, cps, chunk)
    idxi_r = idx_i.astype(jnp.int32).reshape(shape)
    idxj_r = idx_j.astype(jnp.int32).reshape(shape)
    alpha_r = alpha_ij.astype(jnp.float32).reshape(shape)

    partial = _sc_aggregate(x.astype(jnp.float32), alpha_r, idxi_r, idxj_r,
                            n_nodes, feat, n_super, cps, chunk)
    partial = partial.reshape(_NC, n_nodes, feat)
    return _tc_finish(partial[0], partial[1],
                      W.astype(jnp.float32), n_nodes, feat, 400)
